# baseline (speedup 1.0000x reference)
import functools

import jax
import jax.numpy as jnp
from jax import lax
from jax.experimental import pallas as pl
from jax.experimental.pallas import tpu as pltpu
from jax.experimental.pallas import tpu_sc as plsc


@functools.lru_cache(maxsize=None)
def _make_gather(B, J, V, D):
    info = plsc.get_sparse_core_info()
    nw = info.num_cores * info.num_subcores
    J1, J8 = J // 8, 8
    BT, B8 = B // 128, 128
    n_su = J1 * BT
    su_per_w = n_su // nw
    DT = D // 8
    mesh = plsc.VectorSubcoreMesh(core_axis_name="c", subcore_axis_name="s")

    @functools.partial(
        pl.kernel,
        out_type=jax.ShapeDtypeStruct((J, DT, BT, 8, B8), jnp.float32),
        mesh=mesh,
        scratch_types=[
            pltpu.VMEM((2, J8, B8), jnp.int32),
            pltpu.VMEM((2, B8, D), jnp.float32),
            pltpu.VMEM((2, DT, 8, B8), jnp.float32),
            pltpu.SemaphoreType.DMA,
            pltpu.SemaphoreType.DMA,
            pltpu.SemaphoreType.DMA,
            pltpu.SemaphoreType.DMA,
            pltpu.SemaphoreType.DMA,
            pltpu.SemaphoreType.DMA,
        ],
        compiler_params=pltpu.CompilerParams(
            use_tc_tiling_on_sc=False, needs_layout_passes=False),
    )
    def gather_kernel(x4d, w2d, out5d, idx_v, rows_v, tr_v,
                      g0, g1, w0, w1, i0, i1):
        wid = lax.axis_index("s") * info.num_cores + lax.axis_index("c")
        su0 = wid * su_per_w
        gsem = (g0, g1)
        wsem = (w0, w1)
        isem = (i0, i1)
        lane = lax.iota(jnp.int32, 16)

        def idx_load(su, slot):
            jt = su // BT
            bt = su % BT
            return pltpu.make_async_copy(
                x4d.at[jt, bt], idx_v.at[slot], isem[slot])

        def gather(su_slot, j8, rows_slot):
            return pltpu.make_async_copy(
                w2d.at[idx_v.at[su_slot, j8]], rows_v.at[rows_slot],
                gsem[rows_slot])

        def writeback(su, j8, slot):
            jt = su // BT
            bt = su % BT
            return pltpu.make_async_copy(
                tr_v.at[slot], out5d.at[jt * 8 + j8, :, bt], wsem[slot])

        def transpose(slot):
            def t_body(i, carry):
                ft = i // 8
                b0 = (i % 8) * 16
                row = b0 + lane
                for f8 in range(8):
                    col = lax.broadcast(ft * 8 + f8, (16,))
                    v = plsc.load_gather(rows_v.at[slot], [row, col])
                    tr_v[slot, ft, f8, pl.ds(b0, 16)] = v
                return carry
            lax.fori_loop(0, DT * 8, t_body, 0)

        idx_load(su0, 0).start()
        idx_load(su0, 0).wait()
        gather(0, 0, 0).start()

        n2 = su_per_w // 2

        def su_body(k2, carry):
            su_a = su0 + 2 * k2
            su_b = su_a + 1

            for pair, su, su_slot in ((0, su_a, 0), (1, su_b, 1)):
                if pair == 0:
                    idx_load(su_b, 1).start()
                else:
                    @pl.when(k2 < n2 - 1)
                    def _():
                        idx_load(su_b + 1, 0).start()

                for j8 in range(J8):
                    s = j8 % 2
                    gather(su_slot, j8, s).wait()

                    if j8 == J8 - 1:
                        if pair == 0:
                            idx_load(su_b, 1).wait()
                            gather(1, 0, 1 - s).start()
                        else:
                            @pl.when(k2 < n2 - 1)
                            def _():
                                idx_load(su_b + 1, 0).wait()
                                gather(0, 0, 1 - s).start()
                    else:
                        gather(su_slot, j8 + 1, 1 - s).start()

                    if pair == 1 or j8 >= 2:
                        writeback(su, j8, s).wait()
                    else:
                        @pl.when(k2 > 0)
                        def _():
                            writeback(su, j8, s).wait()

                    transpose(s)
                    writeback(su, j8, s).start()
            return carry

        lax.fori_loop(0, n2, su_body, 0)
        writeback(su0 + su_per_w - 1, J8 - 2, 0).wait()
        writeback(su0 + su_per_w - 1, J8 - 1, 1).wait()

    return gather_kernel


def kernel(x, weight):
    B, J = x.shape
    V, D = weight.shape
    x4d = x.reshape(B // 128, 128, J // 8, 8).transpose(2, 0, 3, 1)
    x4d = x4d.astype(jnp.int32)
    out5d = _make_gather(B, J, V, D)(x4d, weight)
    return out5d.transpose(2, 4, 0, 1, 3).reshape(B, J, D)


# speedup vs baseline: 1.1307x; 1.1307x over previous
import functools

import jax
import jax.numpy as jnp
from jax import lax
from jax.experimental import pallas as pl
from jax.experimental.pallas import tpu as pltpu
from jax.experimental.pallas import tpu_sc as plsc


@functools.lru_cache(maxsize=None)
def _make_gather(B, J, V, D):
    info = plsc.get_sparse_core_info()
    nw = info.num_cores * info.num_subcores
    J8 = 8
    BT, B8 = B // 128, 128
    n_su = (J // J8) * BT
    su_per_w = n_su // nw
    n_units = su_per_w * J8
    DT = D // 8
    mesh = plsc.VectorSubcoreMesh(core_axis_name="c", subcore_axis_name="s")

    @functools.partial(
        pl.kernel,
        out_type=jax.ShapeDtypeStruct((J, DT, BT, 8, B8), jnp.float32),
        mesh=mesh,
        scratch_types=[
            pltpu.VMEM((2, J8, B8), jnp.int32),
            pltpu.VMEM((2, B8, D), jnp.float32),
            pltpu.VMEM((DT, 8, B8), jnp.float32),
            pltpu.SemaphoreType.DMA,
            pltpu.SemaphoreType.DMA,
            pltpu.SemaphoreType.DMA,
        ],
        compiler_params=pltpu.CompilerParams(
            use_tc_tiling_on_sc=False, needs_layout_passes=False),
    )
    def gather_kernel(x4d, w2d, out5d, idx_v, rows_v, tr_v, gs, ws, xs):
        wid = lax.axis_index("s") * info.num_cores + lax.axis_index("c")
        su0 = wid * su_per_w
        lane = lax.iota(jnp.int32, 16)
        fidx = [((f0 + lane) // 8, (f0 + lane) % 8) for f0 in range(0, D, 16)]

        def idx_load(k, slot):
            su = su0 + k
            return pltpu.make_async_copy(
                x4d.at[su // BT, su % BT], idx_v.at[slot], xs)

        def gather(u):
            return pltpu.make_async_copy(
                w2d.at[idx_v.at[(u // J8) % 2, u % J8]],
                rows_v.at[u % 2], gs)

        def writeback(u):
            su = su0 + u // J8
            j = (su // BT) * J8 + u % J8
            return pltpu.make_async_copy(
                tr_v, out5d.at[j, :, su % BT], ws)

        def transpose(u):
            s = u % 2

            def t_body(b, bv):
                for i, (ftv, f8v) in enumerate(fidx):
                    for db in range(2):
                        v = rows_v[s, b * 2 + db, pl.ds(i * 16, 16)]
                        plsc.store_scatter(tr_v, [ftv, f8v, bv + db], v)
                return bv + 2

            lax.fori_loop(0, B8 // 2, t_body, lax.broadcast(0, (16,)))

        idx_load(0, 0).start()
        idx_load(0, 0).wait()
        gather(0).start()

        def unit_body(u, carry):
            j8 = u % J8
            k = u // J8

            @pl.when(j8 == 0)
            def _():
                @pl.when(k < su_per_w - 1)
                def _():
                    idx_load(k + 1, (k + 1) % 2).start()

            gather(u).wait()

            @pl.when((j8 == J8 - 1) & (u < n_units - 1))
            def _():
                idx_load(k + 1, (k + 1) % 2).wait()

            @pl.when(u < n_units - 1)
            def _():
                gather(u + 1).start()

            @pl.when(u > 0)
            def _():
                writeback(u - 1).wait()

            transpose(u)
            writeback(u).start()
            return carry

        lax.fori_loop(0, n_units, unit_body, 0)
        writeback(n_units - 1).wait()

    return gather_kernel


def kernel(x, weight):
    B, J = x.shape
    V, D = weight.shape
    x4d = x.reshape(B // 128, 128, J // 8, 8).transpose(2, 0, 3, 1)
    x4d = x4d.astype(jnp.int32)
    out5d = _make_gather(B, J, V, D)(x4d, weight)
    return out5d.transpose(2, 4, 0, 1, 3).reshape(B, J, D)


# speedup vs baseline: 1.4938x; 1.3211x over previous
import functools

import jax
import jax.numpy as jnp
from jax import lax
from jax.experimental import pallas as pl
from jax.experimental.pallas import tpu as pltpu
from jax.experimental.pallas import tpu_sc as plsc


@functools.lru_cache(maxsize=None)
def _make_gather(B, J, V, D):
    info = plsc.get_sparse_core_info()
    nw = info.num_cores * info.num_subcores
    J8 = 8
    BT, B8 = B // 128, 128
    n_su = (J // J8) * BT
    su_per_w = n_su // nw
    n_units = su_per_w * J8
    DT = D // 8
    mesh = plsc.VectorSubcoreMesh(core_axis_name="c", subcore_axis_name="s")

    @functools.partial(
        pl.kernel,
        out_type=jax.ShapeDtypeStruct((J, DT, BT, 8, B8), jnp.float32),
        mesh=mesh,
        scratch_types=[
            pltpu.VMEM((2, J8, B8), jnp.int32),
            pltpu.VMEM((2, B8, D), jnp.float32),
            pltpu.VMEM((DT, 8, B8), jnp.float32),
            pltpu.SemaphoreType.DMA,
            pltpu.SemaphoreType.DMA,
            pltpu.SemaphoreType.DMA,
        ],
        compiler_params=pltpu.CompilerParams(
            use_tc_tiling_on_sc=False, needs_layout_passes=False),
    )
    def gather_kernel(x4d, w2d, out5d, idx_v, rows_v, tr_v, gs, ws, xs):
        wid = lax.axis_index("s") * info.num_cores + lax.axis_index("c")
        su0 = wid * su_per_w
        lane = lax.iota(jnp.int32, 16)
        fidx = [((f0 + lane) // 8, (f0 + lane) % 8) for f0 in range(0, D, 16)]

        def idx_load(k, slot):
            su = su0 + k
            return pltpu.make_async_copy(
                x4d.at[su // BT, su % BT], idx_v.at[slot], xs)

        def gather(u):
            return pltpu.make_async_copy(
                w2d.at[idx_v.at[(u // J8) % 2, u % J8]],
                rows_v.at[u % 2], gs)

        def writeback(u):
            su = su0 + u // J8
            j = (su // BT) * J8 + u % J8
            return pltpu.make_async_copy(
                tr_v, out5d.at[j, :, su % BT], ws)

        def transpose_s(s):
            @plsc.parallel_loop(0, B8 // 2, unroll=4)
            def t_body(b):
                bv = lax.broadcast(b * 2, (16,))
                for i, (ftv, f8v) in enumerate(fidx):
                    for db in range(2):
                        v = rows_v[s, b * 2 + db, pl.ds(i * 16, 16)]
                        plsc.store_scatter(tr_v, [ftv, f8v, bv + db], v)

        def transpose(u):
            @pl.when(u % 2 == 0)
            def _():
                transpose_s(0)

            @pl.when(u % 2 == 1)
            def _():
                transpose_s(1)

        idx_load(0, 0).start()
        idx_load(0, 0).wait()
        gather(0).start()

        def unit_body(u, carry):
            j8 = u % J8
            k = u // J8

            @pl.when(j8 == 0)
            def _():
                @pl.when(k < su_per_w - 1)
                def _():
                    idx_load(k + 1, (k + 1) % 2).start()

            gather(u).wait()

            @pl.when((j8 == J8 - 1) & (u < n_units - 1))
            def _():
                idx_load(k + 1, (k + 1) % 2).wait()

            @pl.when(u < n_units - 1)
            def _():
                gather(u + 1).start()

            @pl.when(u > 0)
            def _():
                writeback(u - 1).wait()

            transpose(u)
            writeback(u).start()
            return carry

        lax.fori_loop(0, n_units, unit_body, 0)
        writeback(n_units - 1).wait()

    return gather_kernel


def kernel(x, weight):
    B, J = x.shape
    V, D = weight.shape
    x4d = x.reshape(B // 128, 128, J // 8, 8).transpose(2, 0, 3, 1)
    x4d = x4d.astype(jnp.int32)
    out5d = _make_gather(B, J, V, D)(x4d, weight)
    return out5d.transpose(2, 4, 0, 1, 3).reshape(B, J, D)


# speedup vs baseline: 3.5897x; 2.4030x over previous
import functools

import jax
import jax.numpy as jnp
from jax import lax
from jax.experimental import pallas as pl
from jax.experimental.pallas import tpu as pltpu
from jax.experimental.pallas import tpu_sc as plsc


@functools.lru_cache(maxsize=None)
def _make_gather(B, J, V, D):
    info = plsc.get_sparse_core_info()
    nw = info.num_cores * info.num_subcores
    J8 = 8
    BT, B8 = B // 128, 128
    n_su = (J // J8) * BT
    su_per_w = n_su // nw
    n_units = su_per_w * J8
    DT = D // 8
    mesh = plsc.VectorSubcoreMesh(core_axis_name="c", subcore_axis_name="s")

    @functools.partial(
        pl.kernel,
        out_type=jax.ShapeDtypeStruct((J, DT, BT, 8, B8), jnp.float32),
        mesh=mesh,
        scratch_types=[
            pltpu.VMEM((2, J8, B8), jnp.int32),
            pltpu.VMEM((2, B8, D), jnp.float32),
            pltpu.VMEM((DT, 8, B8 + 1), jnp.float32),
            pltpu.SemaphoreType.DMA,
            pltpu.SemaphoreType.DMA,
            pltpu.SemaphoreType.DMA,
        ],
        compiler_params=pltpu.CompilerParams(
            use_tc_tiling_on_sc=False, needs_layout_passes=False),
    )
    def gather_kernel(x4d, w2d, out5d, idx_v, rows_v, tr_v, gs, ws, xs):
        wid = lax.axis_index("s") * info.num_cores + lax.axis_index("c")
        su0 = wid * su_per_w
        lane = lax.iota(jnp.int32, 16)
        fidx = [((f0 + lane) // 8, (f0 + lane) % 8) for f0 in range(0, D, 16)]

        def idx_load(k, slot):
            su = su0 + k
            return pltpu.make_async_copy(
                x4d.at[su // BT, su % BT], idx_v.at[slot], xs)

        def gather(u):
            return pltpu.make_async_copy(
                w2d.at[idx_v.at[(u // J8) % 2, u % J8]],
                rows_v.at[u % 2], gs)

        def writeback(u):
            su = su0 + u // J8
            j = (su // BT) * J8 + u % J8
            return pltpu.make_async_copy(
                tr_v.at[:, :, pl.ds(0, B8)], out5d.at[j, :, su % BT], ws)

        def transpose_s(s):
            @plsc.parallel_loop(0, B8 // 2, unroll=4)
            def t_body(b):
                bv = lax.broadcast(b * 2, (16,))
                for i, (ftv, f8v) in enumerate(fidx):
                    for db in range(2):
                        v = rows_v[s, b * 2 + db, pl.ds(i * 16, 16)]
                        plsc.store_scatter(tr_v, [ftv, f8v, bv + db], v)

        def transpose(u):
            @pl.when(u % 2 == 0)
            def _():
                transpose_s(0)

            @pl.when(u % 2 == 1)
            def _():
                transpose_s(1)

        idx_load(0, 0).start()
        idx_load(0, 0).wait()
        gather(0).start()

        def unit_body(u, carry):
            j8 = u % J8
            k = u // J8

            @pl.when(j8 == 0)
            def _():
                @pl.when(k < su_per_w - 1)
                def _():
                    idx_load(k + 1, (k + 1) % 2).start()

            gather(u).wait()

            @pl.when((j8 == J8 - 1) & (u < n_units - 1))
            def _():
                idx_load(k + 1, (k + 1) % 2).wait()

            @pl.when(u < n_units - 1)
            def _():
                gather(u + 1).start()

            @pl.when(u > 0)
            def _():
                writeback(u - 1).wait()

            transpose(u)
            writeback(u).start()
            return carry

        lax.fori_loop(0, n_units, unit_body, 0)
        writeback(n_units - 1).wait()

    return gather_kernel


def kernel(x, weight):
    B, J = x.shape
    V, D = weight.shape
    x4d = x.reshape(B // 128, 128, J // 8, 8).transpose(2, 0, 3, 1)
    x4d = x4d.astype(jnp.int32)
    out5d = _make_gather(B, J, V, D)(x4d, weight)
    return out5d.transpose(2, 4, 0, 1, 3).reshape(B, J, D)


# speedup vs baseline: 3.5909x; 1.0003x over previous
import functools

import jax
import jax.numpy as jnp
from jax import lax
from jax.experimental import pallas as pl
from jax.experimental.pallas import tpu as pltpu
from jax.experimental.pallas import tpu_sc as plsc


@functools.lru_cache(maxsize=None)
def _make_gather(B, J, V, D):
    info = plsc.get_sparse_core_info()
    nw = info.num_cores * info.num_subcores
    J8 = 8
    BT, B8 = B // 128, 128
    n_su = (J // J8) * BT
    su_per_w = n_su // nw
    n_units = su_per_w * J8
    DT = D // 8
    mesh = plsc.VectorSubcoreMesh(core_axis_name="c", subcore_axis_name="s")

    @functools.partial(
        pl.kernel,
        out_type=jax.ShapeDtypeStruct((J, DT, BT, 8, B8), jnp.float32),
        mesh=mesh,
        scratch_types=[
            pltpu.VMEM((2, J8, B8), jnp.int32),
            pltpu.VMEM((2, B8, D), jnp.float32),
            pltpu.VMEM((2, DT, 8, B8 + 1), jnp.float32),
            pltpu.SemaphoreType.DMA,
            pltpu.SemaphoreType.DMA,
            pltpu.SemaphoreType.DMA,
            pltpu.SemaphoreType.DMA,
        ],
        compiler_params=pltpu.CompilerParams(
            use_tc_tiling_on_sc=False, needs_layout_passes=False),
    )
    def gather_kernel(x4d, w2d, out5d, idx_v, rows_v, tr_v, gs, ws0, ws1, xs):
        wid = lax.axis_index("s") * info.num_cores + lax.axis_index("c")
        su0 = wid * su_per_w
        lane = lax.iota(jnp.int32, 16)
        fidx = [((f0 + lane) // 8, (f0 + lane) % 8) for f0 in range(0, D, 16)]

        def idx_load(k, slot):
            su = su0 + k
            return pltpu.make_async_copy(
                x4d.at[su // BT, su % BT], idx_v.at[slot], xs)

        def gather(u):
            return pltpu.make_async_copy(
                w2d.at[idx_v.at[(u // J8) % 2, u % J8]],
                rows_v.at[u % 2], gs)

        wsem = (ws0, ws1)

        def writeback(u, p):
            su = su0 + u // J8
            j = (su // BT) * J8 + u % J8
            return pltpu.make_async_copy(
                tr_v.at[p, :, :, pl.ds(0, B8)], out5d.at[j, :, su % BT],
                wsem[p])

        def transpose_s(p):
            @plsc.parallel_loop(0, B8 // 2, unroll=8)
            def t_body(b):
                bv = lax.broadcast(b * 2, (16,))
                for i, (ftv, f8v) in enumerate(fidx):
                    for db in range(2):
                        v = rows_v[p, b * 2 + db, pl.ds(i * 16, 16)]
                        plsc.store_scatter(tr_v.at[p], [ftv, f8v, bv + db], v)

        idx_load(0, 0).start()
        idx_load(0, 0).wait()
        gather(0).start()

        def pair_body(k2, carry):
            for a in (0, 1):
                u = 2 * k2 + a
                j8 = u % J8
                k = u // J8

                @pl.when(j8 == 0)
                def _():
                    @pl.when(k < su_per_w - 1)
                    def _():
                        idx_load(k + 1, (k + 1) % 2).start()

                gather(u).wait()

                @pl.when((j8 == J8 - 1) & (u < n_units - 1))
                def _():
                    idx_load(k + 1, (k + 1) % 2).wait()

                @pl.when(u < n_units - 1)
                def _():
                    gather(u + 1).start()

                if a == 0:
                    @pl.when(k2 > 0)
                    def _():
                        writeback(u, a).wait()
                else:
                    @pl.when(k2 > 0)
                    def _():
                        writeback(u, a).wait()

                transpose_s(a)
                writeback(u, a).start()
            return carry

        lax.fori_loop(0, n_units // 2, pair_body, 0)
        writeback(n_units - 2, 0).wait()
        writeback(n_units - 1, 1).wait()

    return gather_kernel


def kernel(x, weight):
    B, J = x.shape
    V, D = weight.shape
    x4d = x.reshape(B // 128, 128, J // 8, 8).transpose(2, 0, 3, 1)
    x4d = x4d.astype(jnp.int32)
    out5d = _make_gather(B, J, V, D)(x4d, weight)
    return out5d.transpose(2, 4, 0, 1, 3).reshape(B, J, D)
